# baseline (speedup 1.0000x reference)
import functools

import jax
import jax.numpy as jnp
from jax import lax
from jax.experimental import pallas as pl
from jax.experimental.pallas import tpu as pltpu
from jax.experimental.pallas import tpu_sc as plsc

_NUM_CORES = 2
_NUM_SUBCORES = 16
_NUM_WORKERS = _NUM_CORES * _NUM_SUBCORES
_SRC_ROWS = 64
_LANES = 16
_BIG = 1 << 30


def kernel(token_type_ids, token_type_embeddings):
    batch, seq_len = token_type_ids.shape
    vocab, hidden = token_type_embeddings.shape
    n_rows = batch * seq_len
    rows_per_worker = n_rows // _NUM_WORKERS
    n_groups = rows_per_worker // _LANES
    n_chunks = rows_per_worker // _SRC_ROWS

    ids_flat = token_type_ids.reshape(n_rows).astype(jnp.int32)
    mesh = plsc.VectorSubcoreMesh(core_axis_name="c", subcore_axis_name="s")

    @functools.partial(
        pl.kernel,
        mesh=mesh,
        out_type=jax.ShapeDtypeStruct((n_rows, hidden), jnp.float32),
        scratch_types=[
            pltpu.VMEM((rows_per_worker,), jnp.int32),
            pltpu.VMEM((n_chunks, _SRC_ROWS), jnp.int32),
            pltpu.VMEM((n_chunks, _SRC_ROWS), jnp.int32),
            pltpu.VMEM((_SRC_ROWS, hidden), jnp.float32),
            pltpu.VMEM((_SRC_ROWS, hidden), jnp.float32),
            pltpu.SemaphoreType.DMA,
        ],
    )
    def emb(table_hbm, ids_hbm, out_hbm, ids_v, idx0_v, idx1_v,
            w0_buf, w1_buf, sem):
        wid = lax.axis_index("s") * _NUM_CORES + lax.axis_index("c")
        base = wid * rows_per_worker

        pltpu.sync_copy(ids_hbm.at[pl.ds(base, rows_per_worker)], ids_v)

        iota16 = lax.iota(jnp.int32, _LANES)
        bigv = jnp.full((_LANES,), _BIG, jnp.int32)

        def last_seen(g, carry):
            l0, l1 = carry
            off = pl.multiple_of(g * _LANES, 8)
            ids16 = ids_v[pl.ds(off, _LANES)]
            rows16 = iota16 + (base + g * _LANES)
            m1 = ids16
            m0 = 1 - ids16
            return m0 * rows16 + m1 * l0, m1 * rows16 + m0 * l1

        l0f, l1f = lax.fori_loop(0, n_groups, last_seen, (bigv, bigv))

        f0 = l0f[0]
        f1 = l1f[0]
        for l in range(1, _LANES):
            f0 = jnp.minimum(f0, l0f[l])
            f1 = jnp.minimum(f1, l1f[l])
        has0 = jnp.minimum(_BIG - f0, 1)
        has1 = jnp.minimum(_BIG - f1, 1)
        init0 = jnp.minimum(l0f, f0)
        init1 = jnp.minimum(l1f, f1)

        def build(g, carry):
            l0, l1 = carry
            off = pl.multiple_of(g * _LANES, 8)
            ids16 = ids_v[pl.ds(off, _LANES)]
            rows16 = iota16 + (base + g * _LANES)
            m1 = ids16
            m0 = 1 - ids16
            l0 = m0 * rows16 + m1 * l0
            l1 = m1 * rows16 + m0 * l1
            r = g >> 2
            col = (g & 3) * _LANES
            idx0_v[r, pl.ds(col, _LANES)] = l0
            idx1_v[r, pl.ds(col, _LANES)] = l1
            return l0, l1

        lax.fori_loop(0, n_groups, build, (init0, init1))

        def fill(buf):
            row0 = [buf[0, pl.ds(j * _LANES, _LANES)]
                    for j in range(hidden // _LANES)]

            def rep(r, c, buf=buf, row0=row0):
                for j in range(hidden // _LANES):
                    buf[r, pl.ds(j * _LANES, _LANES)] = row0[j]
                return c

            lax.fori_loop(1, _SRC_ROWS, rep, 0)

        def fire0(j, c):
            pltpu.async_copy(w0_buf, out_hbm.at[idx0_v.at[j]], sem)
            return c

        def fire1(j, c):
            pltpu.async_copy(w1_buf, out_hbm.at[idx1_v.at[j]], sem)
            return c

        pltpu.sync_copy(table_hbm.at[pl.ds(0, 1)], w0_buf.at[pl.ds(0, 1)])
        fill(w0_buf)
        lax.fori_loop(0, n_chunks * has0, fire0, 0)
        pltpu.sync_copy(table_hbm.at[pl.ds(1, 1)], w1_buf.at[pl.ds(0, 1)])
        fill(w1_buf)
        lax.fori_loop(0, n_chunks * has1, fire1, 0)

        def drain(j, c):
            pltpu.make_async_copy(
                out_hbm.at[pl.ds(0, _SRC_ROWS)], w0_buf, sem).wait()
            return c

        lax.fori_loop(0, n_chunks * (has0 + has1), drain, 0)

    out = emb(token_type_embeddings, ids_flat)
    return out.reshape(batch, seq_len, hidden)


# speedup vs baseline: 1.1746x; 1.1746x over previous
import functools

import jax
import jax.numpy as jnp
from jax import lax
from jax.experimental import pallas as pl
from jax.experimental.pallas import tpu as pltpu
from jax.experimental.pallas import tpu_sc as plsc

_NUM_CORES = 2
_NUM_SUBCORES = 16
_NUM_WORKERS = _NUM_CORES * _NUM_SUBCORES
_CHUNK = 64
_LANES = 16


def kernel(token_type_ids, token_type_embeddings):
    batch, seq_len = token_type_ids.shape
    vocab, hidden = token_type_embeddings.shape
    n_rows = batch * seq_len
    rows_per_worker = n_rows // _NUM_WORKERS
    n_chunks = rows_per_worker // _CHUNK
    half = hidden // 2
    jh = half // _LANES

    ids_flat = token_type_ids.reshape(n_rows).astype(jnp.int32)
    mesh = plsc.VectorSubcoreMesh(core_axis_name="c", subcore_axis_name="s")

    @functools.partial(
        pl.kernel,
        mesh=mesh,
        out_type=jax.ShapeDtypeStruct((n_rows, hidden), jnp.float32),
        scratch_types=[
            pltpu.VMEM((rows_per_worker,), jnp.int32),
            pltpu.VMEM((vocab, hidden), jnp.float32),
            pltpu.VMEM((2, _CHUNK, hidden), jnp.float32),
            pltpu.SemaphoreType.DMA((2,)),
        ],
    )
    def emb(table_hbm, ids_hbm, out_hbm, ids_v, tbl_v, buf3, sems):
        wid = lax.axis_index("s") * _NUM_CORES + lax.axis_index("c")
        base = wid * rows_per_worker

        pltpu.sync_copy(ids_hbm.at[pl.ds(base, rows_per_worker)], ids_v)
        pltpu.sync_copy(table_hbm, tbl_v)

        for p in range(2):
            pltpu.async_copy(out_hbm.at[pl.ds(0, _CHUNK)], buf3.at[p],
                             sems.at[p])

        def chunk_body(c, carry):
            par = c & 1
            pltpu.make_async_copy(
                out_hbm.at[pl.ds(0, _CHUNK)], buf3.at[0],
                sems.at[par]).wait()

            for h in range(2):
                col0 = h * half
                w0 = [tbl_v[0, pl.ds(col0 + j * _LANES, _LANES)]
                      for j in range(jh)]
                w1 = [tbl_v[1, pl.ds(col0 + j * _LANES, _LANES)]
                      for j in range(jh)]

                def group_body(g, cc, w0=w0, w1=w1, col0=col0):
                    off = pl.multiple_of(c * _CHUNK + g * _LANES, 8)
                    m16 = ids_v[pl.ds(off, _LANES)].astype(jnp.float32)
                    for l in range(_LANES):
                        m = jnp.full((_LANES,), m16[l], jnp.float32)
                        om = 1.0 - m
                        row = g * _LANES + l
                        for j in range(jh):
                            buf3[par, row,
                                 pl.ds(col0 + j * _LANES, _LANES)] = (
                                     m * w1[j] + om * w0[j])
                    return cc

                lax.fori_loop(0, _CHUNK // _LANES, group_body, 0)

            pltpu.async_copy(
                buf3.at[par],
                out_hbm.at[pl.ds(base + c * _CHUNK, _CHUNK)],
                sems.at[par])
            return carry

        lax.fori_loop(0, n_chunks, chunk_body, 0)

        for p in range(2):
            pltpu.make_async_copy(
                out_hbm.at[pl.ds(0, _CHUNK)], buf3.at[0],
                sems.at[p]).wait()

    out = emb(token_type_embeddings, ids_flat)
    return out.reshape(batch, seq_len, hidden)


# speedup vs baseline: 1.3817x; 1.1763x over previous
import functools

import jax
import jax.numpy as jnp
from jax import lax
from jax.experimental import pallas as pl
from jax.experimental.pallas import tpu as pltpu
from jax.experimental.pallas import tpu_sc as plsc

_NUM_CORES = 2
_NUM_SUBCORES = 16
_NUM_WORKERS = _NUM_CORES * _NUM_SUBCORES
_CHUNK = 64
_LANES = 16


def kernel(token_type_ids, token_type_embeddings):
    batch, seq_len = token_type_ids.shape
    vocab, hidden = token_type_embeddings.shape
    n_rows = batch * seq_len
    rows_per_worker = n_rows // _NUM_WORKERS
    n_chunks = rows_per_worker // _CHUNK
    half = hidden // 2
    jh = half // _LANES

    ids_flat = token_type_ids.reshape(n_rows).astype(jnp.int32)
    mesh = plsc.VectorSubcoreMesh(core_axis_name="c", subcore_axis_name="s")

    @functools.partial(
        pl.kernel,
        mesh=mesh,
        out_type=jax.ShapeDtypeStruct((n_rows, hidden), jnp.float32),
        scratch_types=[
            pltpu.VMEM((rows_per_worker,), jnp.int32),
            pltpu.VMEM((vocab, hidden), jnp.float32),
            pltpu.VMEM((2, _CHUNK, hidden), jnp.float32),
            pltpu.SemaphoreType.DMA((2,)),
        ],
    )
    def emb(table_hbm, ids_hbm, out_hbm, ids_v, tbl_v, buf3, sems):
        wid = lax.axis_index("s") * _NUM_CORES + lax.axis_index("c")
        base = wid * rows_per_worker

        pltpu.sync_copy(ids_hbm.at[pl.ds(base, rows_per_worker)], ids_v)
        pltpu.sync_copy(table_hbm, tbl_v)

        for p in range(2):
            pltpu.async_copy(out_hbm.at[pl.ds(0, _CHUNK)], buf3.at[p],
                             sems.at[p])

        def chunk_body(c, carry):
            par = c & 1
            pltpu.make_async_copy(
                out_hbm.at[pl.ds(0, _CHUNK)], buf3.at[0],
                sems.at[par]).wait()

            for h in range(2):
                col0 = h * half
                w0 = [tbl_v[0, pl.ds(col0 + j * _LANES, _LANES)]
                      for j in range(jh)]
                d = [tbl_v[1, pl.ds(col0 + j * _LANES, _LANES)] - w0[j]
                     for j in range(jh)]

                def group_body(g, cc, w0=w0, d=d, col0=col0):
                    off = pl.multiple_of(c * _CHUNK + g * _LANES, 8)
                    m16 = ids_v[pl.ds(off, _LANES)].astype(jnp.float32)
                    for l in range(_LANES):
                        m = jnp.full((_LANES,), m16[l], jnp.float32)
                        row = g * _LANES + l
                        for j in range(jh):
                            buf3[par, row,
                                 pl.ds(col0 + j * _LANES, _LANES)] = (
                                     w0[j] + m * d[j])
                    return cc

                lax.fori_loop(0, _CHUNK // _LANES, group_body, 0)

            pltpu.async_copy(
                buf3.at[par],
                out_hbm.at[pl.ds(base + c * _CHUNK, _CHUNK)],
                sems.at[par])
            return carry

        lax.fori_loop(0, n_chunks, chunk_body, 0)

        for p in range(2):
            pltpu.make_async_copy(
                out_hbm.at[pl.ds(0, _CHUNK)], buf3.at[0],
                sems.at[p]).wait()

    out = emb(token_type_embeddings, ids_flat)
    return out.reshape(batch, seq_len, hidden)


# speedup vs baseline: 1.5961x; 1.1552x over previous
import functools

import jax
import jax.numpy as jnp
from jax import lax
from jax.experimental import pallas as pl
from jax.experimental.pallas import tpu as pltpu
from jax.experimental.pallas import tpu_sc as plsc

_NUM_CORES = 2
_NUM_SUBCORES = 16
_NUM_WORKERS = _NUM_CORES * _NUM_SUBCORES
_CHUNK = 64
_LANES = 16


def kernel(token_type_ids, token_type_embeddings):
    batch, seq_len = token_type_ids.shape
    vocab, hidden = token_type_embeddings.shape
    n_rows = batch * seq_len
    rows_per_worker = n_rows // _NUM_WORKERS
    n_chunks = rows_per_worker // _CHUNK
    half = hidden // 2
    jh = half // _LANES

    ids_flat = token_type_ids.reshape(n_rows).astype(jnp.int32)
    mesh = plsc.VectorSubcoreMesh(core_axis_name="c", subcore_axis_name="s")

    @functools.partial(
        pl.kernel,
        mesh=mesh,
        out_type=jax.ShapeDtypeStruct((n_rows, hidden), jnp.float32),
        scratch_types=[
            pltpu.VMEM((rows_per_worker,), jnp.int32),
            pltpu.VMEM((vocab, hidden), jnp.float32),
            pltpu.VMEM((2, _CHUNK, hidden), jnp.float32),
            pltpu.SemaphoreType.DMA((2,)),
        ],
    )
    def emb(table_hbm, ids_hbm, out_hbm, ids_v, tbl_v, buf3, sems):
        wid = lax.axis_index("s") * _NUM_CORES + lax.axis_index("c")
        base = wid * rows_per_worker

        pltpu.sync_copy(ids_hbm.at[pl.ds(base, rows_per_worker)], ids_v)
        pltpu.sync_copy(table_hbm, tbl_v)

        for p in range(2):
            pltpu.async_copy(out_hbm.at[pl.ds(base + p * _CHUNK, _CHUNK)],
                             buf3.at[p], sems.at[p])

        def chunk_body(c, carry):
            par = c & 1
            pltpu.make_async_copy(
                out_hbm.at[pl.ds(0, _CHUNK)], buf3.at[0],
                sems.at[par]).wait()

            for h in range(2):
                col0 = h * half
                w0 = [tbl_v[0, pl.ds(col0 + j * _LANES, _LANES)]
                      for j in range(jh)]
                d = [tbl_v[1, pl.ds(col0 + j * _LANES, _LANES)] - w0[j]
                     for j in range(jh)]

                def group_body(g, cc, w0=w0, d=d, col0=col0):
                    off = pl.multiple_of(c * _CHUNK + g * _LANES, 8)
                    m16 = ids_v[pl.ds(off, _LANES)].astype(jnp.float32)
                    for l in range(_LANES):
                        m = jnp.full((_LANES,), m16[l], jnp.float32)
                        row = g * _LANES + l
                        for j in range(jh):
                            buf3[par, row,
                                 pl.ds(col0 + j * _LANES, _LANES)] = (
                                     w0[j] + m * d[j])
                    return cc

                lax.fori_loop(0, _CHUNK // _LANES, group_body, 0)

            pltpu.async_copy(
                buf3.at[par],
                out_hbm.at[pl.ds(base + c * _CHUNK, _CHUNK)],
                sems.at[par])
            return carry

        lax.fori_loop(0, n_chunks, chunk_body, 0)

        for p in range(2):
            pltpu.make_async_copy(
                out_hbm.at[pl.ds(0, _CHUNK)], buf3.at[0],
                sems.at[p]).wait()

    out = emb(token_type_embeddings, ids_flat)
    return out.reshape(batch, seq_len, hidden)
